# Initial kernel scaffold; baseline (speedup 1.0000x reference)
import jax
import jax.numpy as jnp
from jax.experimental import pallas as pl


def kernel(xyz, R_min, R_max):
    raise NotImplementedError("write your pallas kernel here")


# speedup vs baseline: 21.8247x; 21.8247x over previous
import functools

import numpy as np
import jax
import jax.numpy as jnp
from jax.experimental import pallas as pl

_NUM_GROUP = 64
_GROUP_SIZE = 256


def _round_up(x, m):
    return (x + m - 1) // m * m


def _num_fps_points(num_points):
    np.random.seed(0)
    crop_rate = float(np.random.random())
    down_rate = _GROUP_SIZE / (num_points * crop_rate)
    return int(num_points * down_rate)


def _fps_kernel(x_ref, y_ref, z_ref,
                px_ref, py_ref, pz_ref, cx_ref, cy_ref, cz_ref,
                *, n, npts, ngroup, npad):
    X = x_ref[...]
    Y = y_ref[...]
    Z = z_ref[...]
    B = X.shape[0]
    I = jax.lax.broadcasted_iota(jnp.int32, (B, n), 1)
    Ip = jax.lax.broadcasted_iota(jnp.int32, (B, npad), 1)
    Ig = jax.lax.broadcasted_iota(jnp.int32, (B, ngroup), 1)

    def fps_step(i, dists, far, A, Aacc, Bc, Bacc, Cc, Cacc, Isrc, Iacc, nsent):
        oh = Isrc == far
        cx = jnp.sum(jnp.where(oh, A, 0.0), axis=1, keepdims=True)
        cy = jnp.sum(jnp.where(oh, Bc, 0.0), axis=1, keepdims=True)
        cz = jnp.sum(jnp.where(oh, Cc, 0.0), axis=1, keepdims=True)
        sel = Iacc == i
        Aacc = jnp.where(sel, cx, Aacc)
        Bacc = jnp.where(sel, cy, Bacc)
        Cacc = jnp.where(sel, cz, Cacc)
        dx = A - cx
        d = dx * dx
        dy = Bc - cy
        d = d + dy * dy
        dz = Cc - cz
        d = d + dz * dz
        dists = jnp.minimum(dists, d)
        m = jnp.max(dists, axis=1, keepdims=True)
        far = jnp.min(jnp.where(dists == m, Isrc, nsent), axis=1, keepdims=True)
        return dists, far, Aacc, Bacc, Cacc

    def body1(i, st):
        dists, far, PX, PY, PZ = st
        dists, far, PX, PY, PZ = fps_step(
            i, dists, far, X, PX, Y, PY, Z, PZ, I, Ip, n)
        return dists, far, PX, PY, PZ

    dists0 = jnp.full((B, n), 1e10, jnp.float32)
    far0 = jnp.zeros((B, 1), jnp.int32)
    P0 = jnp.zeros((B, npad), jnp.float32)
    _, _, PX, PY, PZ = jax.lax.fori_loop(
        0, npts, body1, (dists0, far0, P0, P0, P0))
    px_ref[...] = PX
    py_ref[...] = PY
    pz_ref[...] = PZ

    def body2(j, st):
        dists, far, CX, CY, CZ = st
        dists, far, CX, CY, CZ = fps_step(
            j, dists, far, PX, CX, PY, CY, PZ, CZ, Ip, Ig, npad)
        return dists, far, CX, CY, CZ

    dists20 = jnp.where(Ip < npts, jnp.float32(1e10), jnp.float32(-1.0))
    C0 = jnp.zeros((B, ngroup), jnp.float32)
    _, _, CX, CY, CZ = jax.lax.fori_loop(
        0, ngroup, body2, (dists20, far0, C0, C0, C0))
    cx_ref[...] = CX
    cy_ref[...] = CY
    cz_ref[...] = CZ


def _knn_kernel(xr_ref, yr_ref, zr_ref, cx_ref, cy_ref, cz_ref,
                nx_ref, ny_ref, nz_ref, *, npts, npad, gsz):
    XR = xr_ref[...]
    YR = yr_ref[...]
    ZR = zr_ref[...]
    cx = cx_ref[...]
    cy = cy_ref[...]
    cz = cz_ref[...]
    R = XR.shape[0]
    bf = lambda a: a.astype(jnp.bfloat16).astype(jnp.float32)
    XRb, YRb, ZRb = bf(XR), bf(YR), bf(ZR)
    cxb, cyb, czb = bf(cx), bf(cy), bf(cz)
    mm = (cxb * XRb + cyb * YRb) + czb * ZRb
    c2 = (cx * cx + cy * cy) + cz * cz
    p2 = (XR * XR + YR * YR) + ZR * ZR
    D = -2.0 * mm
    D = D + c2
    D = D + p2
    I = jax.lax.broadcasted_iota(jnp.int32, (R, npad), 1)
    Io = jax.lax.broadcasted_iota(jnp.int32, (R, gsz), 1)
    D = jnp.where(I < npts, D, jnp.inf)

    def body(p, st):
        D, NX, NY, NZ = st
        m = jnp.min(D, axis=1, keepdims=True)
        sel = jnp.min(jnp.where(D == m, I, npad), axis=1, keepdims=True)
        oh = I == sel
        gx = jnp.sum(jnp.where(oh, XR, 0.0), axis=1, keepdims=True)
        gy = jnp.sum(jnp.where(oh, YR, 0.0), axis=1, keepdims=True)
        gz = jnp.sum(jnp.where(oh, ZR, 0.0), axis=1, keepdims=True)
        so = Io == p
        NX = jnp.where(so, gx - cx, NX)
        NY = jnp.where(so, gy - cy, NY)
        NZ = jnp.where(so, gz - cz, NZ)
        return jnp.where(oh, jnp.inf, D), NX, NY, NZ

    N0 = jnp.zeros((R, gsz), jnp.float32)
    _, NX, NY, NZ = jax.lax.fori_loop(0, gsz, body, (D, N0, N0, N0))
    nx_ref[...] = NX
    ny_ref[...] = NY
    nz_ref[...] = NZ


def kernel(xyz, R_min, R_max):
    B, N, _ = xyz.shape
    npts = _num_fps_points(N)
    npad = _round_up(npts, 128)
    f32 = jnp.float32
    xyz = xyz * (R_max - R_min) + R_min
    X = xyz[..., 0]
    Y = xyz[..., 1]
    Z = xyz[..., 2]

    fps = pl.pallas_call(
        functools.partial(_fps_kernel, n=N, npts=npts, ngroup=_NUM_GROUP,
                          npad=npad),
        out_shape=[jax.ShapeDtypeStruct((B, npad), f32)] * 3
        + [jax.ShapeDtypeStruct((B, _NUM_GROUP), f32)] * 3,
    )
    PX, PY, PZ, CX, CY, CZ = fps(X, Y, Z)

    R = B * _NUM_GROUP
    XR = jnp.broadcast_to(PX[:, None, :], (B, _NUM_GROUP, npad)).reshape(R, npad)
    YR = jnp.broadcast_to(PY[:, None, :], (B, _NUM_GROUP, npad)).reshape(R, npad)
    ZR = jnp.broadcast_to(PZ[:, None, :], (B, _NUM_GROUP, npad)).reshape(R, npad)
    cxr = CX.reshape(R, 1)
    cyr = CY.reshape(R, 1)
    czr = CZ.reshape(R, 1)

    knn = pl.pallas_call(
        functools.partial(_knn_kernel, npts=npts, npad=npad, gsz=_GROUP_SIZE),
        out_shape=[jax.ShapeDtypeStruct((R, _GROUP_SIZE), f32)] * 3,
    )
    NX, NY, NZ = knn(XR, YR, ZR, cxr, cyr, czr)

    neighborhood = jnp.stack([NX, NY, NZ], axis=-1).reshape(
        B, _NUM_GROUP, _GROUP_SIZE, 3)
    center = jnp.stack([CX, CY, CZ], axis=-1).reshape(B, _NUM_GROUP, 3)
    return (neighborhood, center)


# speedup vs baseline: 38.1445x; 1.7478x over previous
import functools

import numpy as np
import jax
import jax.numpy as jnp
from jax.experimental import pallas as pl

_NUM_GROUP = 64
_GROUP_SIZE = 256


def _round_up(x, m):
    return (x + m - 1) // m * m


def _num_fps_points(num_points):
    np.random.seed(0)
    crop_rate = float(np.random.random())
    down_rate = _GROUP_SIZE / (num_points * crop_rate)
    return int(num_points * down_rate)


def _fps_kernel(x_ref, y_ref, z_ref,
                px_ref, py_ref, pz_ref, cx_ref, cy_ref, cz_ref,
                *, n, npts, ngroup, npad):
    X = x_ref[...]
    Y = y_ref[...]
    Z = z_ref[...]
    B = X.shape[0]
    I = jax.lax.broadcasted_iota(jnp.int32, (B, n), 1)
    Ip = jax.lax.broadcasted_iota(jnp.int32, (B, npad), 1)
    Ig = jax.lax.broadcasted_iota(jnp.int32, (B, ngroup), 1)

    def fps_step(i, dists, far, A, Aacc, Bc, Bacc, Cc, Cacc, Isrc, Iacc, nsent):
        oh = Isrc == far
        cx = jnp.sum(jnp.where(oh, A, 0.0), axis=1, keepdims=True)
        cy = jnp.sum(jnp.where(oh, Bc, 0.0), axis=1, keepdims=True)
        cz = jnp.sum(jnp.where(oh, Cc, 0.0), axis=1, keepdims=True)
        sel = Iacc == i
        Aacc = jnp.where(sel, cx, Aacc)
        Bacc = jnp.where(sel, cy, Bacc)
        Cacc = jnp.where(sel, cz, Cacc)
        dx = A - cx
        d = dx * dx
        dy = Bc - cy
        d = d + dy * dy
        dz = Cc - cz
        d = d + dz * dz
        dists = jnp.minimum(dists, d)
        m = jnp.max(dists, axis=1, keepdims=True)
        far = jnp.min(jnp.where(dists == m, Isrc, nsent), axis=1, keepdims=True)
        return dists, far, Aacc, Bacc, Cacc

    def body1(i, st):
        dists, far, PX, PY, PZ = st
        dists, far, PX, PY, PZ = fps_step(
            i, dists, far, X, PX, Y, PY, Z, PZ, I, Ip, n)
        return dists, far, PX, PY, PZ

    dists0 = jnp.full((B, n), 1e10, jnp.float32)
    far0 = jnp.zeros((B, 1), jnp.int32)
    P0 = jnp.zeros((B, npad), jnp.float32)
    _, _, PX, PY, PZ = jax.lax.fori_loop(
        0, npts, body1, (dists0, far0, P0, P0, P0))
    px_ref[...] = PX
    py_ref[...] = PY
    pz_ref[...] = PZ

    def body2(j, st):
        dists, far, CX, CY, CZ = st
        dists, far, CX, CY, CZ = fps_step(
            j, dists, far, PX, CX, PY, CY, PZ, CZ, Ip, Ig, npad)
        return dists, far, CX, CY, CZ

    dists20 = jnp.where(Ip < npts, jnp.float32(1e10), jnp.float32(-1.0))
    C0 = jnp.zeros((B, ngroup), jnp.float32)
    _, _, CX, CY, CZ = jax.lax.fori_loop(
        0, ngroup, body2, (dists20, far0, C0, C0, C0))
    cx_ref[...] = CX
    cy_ref[...] = CY
    cz_ref[...] = CZ


def _knn_sort_kernel(xt_ref, yt_ref, zt_ref, cx_ref, cy_ref, cz_ref,
                     io_ref, *, npts, npad, gsz):
    Xt = xt_ref[...]
    Yt = yt_ref[...]
    Zt = zt_ref[...]
    cx = cx_ref[...]
    cy = cy_ref[...]
    cz = cz_ref[...]
    bf = lambda a: a.astype(jnp.bfloat16).astype(jnp.float32)
    mm = (bf(cx) * bf(Xt) + bf(cy) * bf(Yt)) + bf(cz) * bf(Zt)
    c2 = (cx * cx + cy * cy) + cz * cz
    p2 = (Xt * Xt + Yt * Yt) + Zt * Zt
    key = -2.0 * mm
    key = key + c2
    key = key + p2

    row = jax.lax.broadcasted_iota(jnp.int32, key.shape, 0)
    key = jnp.where(row < npts, key, jnp.inf)
    idx = row
    logn = npad.bit_length() - 1
    for kk in range(1, logn + 1):
        asc = (row & (1 << kk)) == 0
        for jj in range(kk - 1, -1, -1):
            s = 1 << jj
            lo = (row & s) == 0
            kt = jnp.where(lo, jnp.roll(key, -s, axis=0),
                           jnp.roll(key, s, axis=0))
            it = jnp.where(lo, jnp.roll(idx, -s, axis=0),
                           jnp.roll(idx, s, axis=0))
            take_smaller = asc == lo
            theirs_smaller = (kt < key) | ((kt == key) & (it < idx))
            use_theirs = take_smaller == theirs_smaller
            key = jnp.where(use_theirs, kt, key)
            idx = jnp.where(use_theirs, it, idx)
    io_ref[...] = idx[:gsz]


def _gather_kernel(xr_ref, yr_ref, zr_ref, cx_ref, cy_ref, cz_ref, idx_ref,
                   nx_ref, ny_ref, nz_ref, *, npad):
    IDX = idx_ref[...]
    local = IDX & 127
    chunk = IDX >> 7
    cx = cx_ref[...]
    cy = cy_ref[...]
    cz = cz_ref[...]
    gx = gy = gz = None
    for c in range(npad // 128):
        sx = jnp.take_along_axis(xr_ref[:, c * 128:(c + 1) * 128], local, axis=1)
        sy = jnp.take_along_axis(yr_ref[:, c * 128:(c + 1) * 128], local, axis=1)
        sz = jnp.take_along_axis(zr_ref[:, c * 128:(c + 1) * 128], local, axis=1)
        if gx is None:
            gx, gy, gz = sx, sy, sz
        else:
            m = chunk == c
            gx = jnp.where(m, sx, gx)
            gy = jnp.where(m, sy, gy)
            gz = jnp.where(m, sz, gz)
    nx_ref[...] = gx - cx
    ny_ref[...] = gy - cy
    nz_ref[...] = gz - cz


def kernel(xyz, R_min, R_max):
    B, N, _ = xyz.shape
    npts = _num_fps_points(N)
    npad = max(128, 1 << (npts - 1).bit_length())
    f32 = jnp.float32
    xyz = xyz * (R_max - R_min) + R_min
    X = xyz[..., 0]
    Y = xyz[..., 1]
    Z = xyz[..., 2]

    fps = pl.pallas_call(
        functools.partial(_fps_kernel, n=N, npts=npts, ngroup=_NUM_GROUP,
                          npad=npad),
        out_shape=[jax.ShapeDtypeStruct((B, npad), f32)] * 3
        + [jax.ShapeDtypeStruct((B, _NUM_GROUP), f32)] * 3,
    )
    PX, PY, PZ, CX, CY, CZ = fps(X, Y, Z)

    R = B * _NUM_GROUP
    XRt = jnp.broadcast_to(PX.T[:, :, None], (npad, B, _NUM_GROUP)).reshape(npad, R)
    YRt = jnp.broadcast_to(PY.T[:, :, None], (npad, B, _NUM_GROUP)).reshape(npad, R)
    ZRt = jnp.broadcast_to(PZ.T[:, :, None], (npad, B, _NUM_GROUP)).reshape(npad, R)
    cxt = CX.reshape(1, R)
    cyt = CY.reshape(1, R)
    czt = CZ.reshape(1, R)

    knn_sort = pl.pallas_call(
        functools.partial(_knn_sort_kernel, npts=npts, npad=npad,
                          gsz=_GROUP_SIZE),
        out_shape=jax.ShapeDtypeStruct((_GROUP_SIZE, R), jnp.int32),
    )
    IO = knn_sort(XRt, YRt, ZRt, cxt, cyt, czt)
    IDXt = IO.T.reshape(R, _GROUP_SIZE)

    XR = jnp.broadcast_to(PX[:, None, :], (B, _NUM_GROUP, npad)).reshape(R, npad)
    YR = jnp.broadcast_to(PY[:, None, :], (B, _NUM_GROUP, npad)).reshape(R, npad)
    ZR = jnp.broadcast_to(PZ[:, None, :], (B, _NUM_GROUP, npad)).reshape(R, npad)
    cxr = CX.reshape(R, 1)
    cyr = CY.reshape(R, 1)
    czr = CZ.reshape(R, 1)

    gather = pl.pallas_call(
        functools.partial(_gather_kernel, npad=npad),
        out_shape=[jax.ShapeDtypeStruct((R, _GROUP_SIZE), f32)] * 3,
    )
    NX, NY, NZ = gather(XR, YR, ZR, cxr, cyr, czr, IDXt)

    neighborhood = jnp.stack([NX, NY, NZ], axis=-1).reshape(
        B, _NUM_GROUP, _GROUP_SIZE, 3)
    center = jnp.stack([CX, CY, CZ], axis=-1).reshape(B, _NUM_GROUP, 3)
    return (neighborhood, center)
